# Initial kernel scaffold; baseline (speedup 1.0000x reference)
import jax
import jax.numpy as jnp
from jax.experimental import pallas as pl


def kernel(x, edge_index, W, b, ln_gamma, ln_beta):
    raise NotImplementedError("write your pallas kernel here")


# speedup vs baseline: 7.1879x; 7.1879x over previous
import functools
import math

import jax
import jax.numpy as jnp
from jax import lax
from jax.experimental import pallas as pl
from jax.experimental.pallas import tpu as pltpu
from jax.experimental.pallas import tpu_sc as plsc

N = 10000
E = 320000
D = 128
DH = D // 2

NC = 2
NS = 16
NW = NC * NS

CH = 125
CN = (E // CH) // NS
EW = E // NW
ROWS_PER_TILE = N // NS
WB = ROWS_PER_TILE // CH

L = 16


def _make_sc_kernel():
    mesh = plsc.VectorSubcoreMesh(core_axis_name="c", subcore_axis_name="s")

    @functools.partial(
        pl.kernel,
        out_type=[
            jax.ShapeDtypeStruct((NC, N, DH), jnp.float32),
            jax.ShapeDtypeStruct((NW, N), jnp.float32),
        ],
        mesh=mesh,
        compiler_params=pltpu.CompilerParams(use_tc_tiling_on_sc=False,
                                             needs_layout_passes=False),
        scratch_types=[
            pltpu.VMEM((CN, CH), jnp.int32),
            pltpu.VMEM((CN, CH), jnp.int32),
            pltpu.VMEM((EW,), jnp.int32),
            pltpu.VMEM((CH, DH), jnp.float32),
            pltpu.VMEM((N,), jnp.float32),
            pltpu.VMEM_SHARED((N, DH), jnp.float32),
            pltpu.SemaphoreType.DMA,
        ],
    )
    def sc_kernel(x0_hbm, x1_hbm, src2d, dst2d, dst1d, agg_out, deg_out,
                  src_v, dst_v, dstf_v, rows_v, deg_v,
                  agg_sh, gsem):
        cid = lax.axis_index("c")
        sid = lax.axis_index("s")
        w = cid * NS + sid
        base = sid * CN
        rbase = sid * ROWS_PER_TILE
        zvec = jnp.zeros((L,), jnp.float32)
        ovec = jnp.ones((L,), jnp.float32)

        pltpu.sync_copy(src2d.at[pl.ds(base, CN)], src_v)
        pltpu.sync_copy(dst2d.at[pl.ds(base, CN)], dst_v)
        pltpu.sync_copy(dst1d.at[pl.ds(w * EW, EW)], dstf_v)

        def fill_rows(i, _):
            for k in range(DH // L):
                rows_v[i, pl.ds(k * L, L)] = zvec
            return 0
        lax.fori_loop(0, CH, fill_rows, 0)

        def fill_deg(i, _):
            deg_v[pl.ds(i * L, L)] = zvec
            return 0
        lax.fori_loop(0, N // L, fill_deg, 0)

        for t in range(WB):
            pltpu.sync_copy(rows_v, agg_sh.at[pl.ds(rbase + t * CH, CH)])
        plsc.subcore_barrier()

        def chunk_c0(j, _):
            pltpu.async_copy(x0_hbm.at[src_v.at[j]], rows_v, gsem).wait()
            pltpu.sync_copy(rows_v, agg_sh.at[dst_v.at[j]], add=True)
            return 0

        def chunk_c1(j, _):
            pltpu.async_copy(x1_hbm.at[src_v.at[j]], rows_v, gsem).wait()
            pltpu.sync_copy(rows_v, agg_sh.at[dst_v.at[j]], add=True)
            return 0

        @pl.when(cid == 0)
        def _():
            lax.fori_loop(0, CN, chunk_c0, 0)

        @pl.when(cid == 1)
        def _():
            lax.fori_loop(0, CN, chunk_c1, 0)

        def count(i, _):
            idx = dstf_v[pl.ds(i * L, L)]
            plsc.addupdate_scatter(deg_v, [idx], ovec)
            return 0
        lax.fori_loop(0, EW // L, count, 0)

        plsc.subcore_barrier()

        for t in range(WB):
            pltpu.sync_copy(agg_sh.at[pl.ds(rbase + t * CH, CH)],
                            agg_out.at[cid, pl.ds(rbase + t * CH, CH)])
        pltpu.sync_copy(deg_v, deg_out.at[w])

    return sc_kernel


_sc_kernel = _make_sc_kernel()


def _tc_body(agg_ref, deg_ref, x_ref, w_ref, b_ref, g_ref, bt_ref, out_ref):
    a = jnp.concatenate([agg_ref[0], agg_ref[1]], axis=1)
    dg = jnp.sum(deg_ref[...], axis=1, keepdims=True)
    dg = jnp.maximum(dg, 1.0)
    an = a / dg
    h = lax.dot_general(an, w_ref[...], (((1,), (0,)), ((), ())),
                        preferred_element_type=jnp.float32,
                        precision=lax.Precision.HIGHEST)
    o = h + b_ref[...] + x_ref[...]
    mu = jnp.mean(o, axis=1, keepdims=True)
    c = o - mu
    var = jnp.mean(c * c, axis=1, keepdims=True)
    y = c * lax.rsqrt(var + 1e-5) * g_ref[...] + bt_ref[...]
    out_ref[...] = 0.5 * y * (1.0 + lax.erf(y * (1.0 / math.sqrt(2.0))))


BN = 1000


def _tc_tail(agg_p, deg_p, x, W, b, g, bt):
    grid = (N // BN,)
    return pl.pallas_call(
        _tc_body,
        grid=grid,
        in_specs=[
            pl.BlockSpec((NC, BN, DH), lambda i: (0, i, 0)),
            pl.BlockSpec((BN, NW), lambda i: (i, 0)),
            pl.BlockSpec((BN, D), lambda i: (i, 0)),
            pl.BlockSpec((D, D), lambda i: (0, 0)),
            pl.BlockSpec((1, D), lambda i: (0, 0)),
            pl.BlockSpec((1, D), lambda i: (0, 0)),
            pl.BlockSpec((1, D), lambda i: (0, 0)),
        ],
        out_specs=pl.BlockSpec((BN, D), lambda i: (i, 0)),
        out_shape=jax.ShapeDtypeStruct((N, D), jnp.float32),
    )(agg_p, deg_p, x, W, b, g, bt)


@jax.jit
def kernel(x, edge_index, W, b, ln_gamma, ln_beta):
    src2d = edge_index[0].reshape(E // CH, CH)
    dst2d = edge_index[1].reshape(E // CH, CH)
    x0 = x[:, :DH]
    x1 = x[:, DH:]
    agg_p, deg_p = _sc_kernel(x0, x1, src2d, dst2d, edge_index[1])
    return _tc_tail(agg_p, deg_p.T, x, W,
                    b.reshape(1, D), ln_gamma.reshape(1, D),
                    ln_beta.reshape(1, D))


# speedup vs baseline: 10.3164x; 1.4352x over previous
import functools
import math

import jax
import jax.numpy as jnp
from jax import lax
from jax.experimental import pallas as pl
from jax.experimental.pallas import tpu as pltpu
from jax.experimental.pallas import tpu_sc as plsc

N = 10000
E = 320000
D = 128
DH = D // 2

NC = 2
NS = 16
NW = NC * NS

CH = 125
CN = (E // CH) // NS
EW = E // NW
ROWS_PER_TILE = N // NS
WB = ROWS_PER_TILE // CH

L = 16


def _make_sc_kernel():
    mesh = plsc.VectorSubcoreMesh(core_axis_name="c", subcore_axis_name="s")

    @functools.partial(
        pl.kernel,
        out_type=[
            jax.ShapeDtypeStruct((NC, N, DH), jnp.float32),
            jax.ShapeDtypeStruct((NW, N), jnp.float32),
        ],
        mesh=mesh,
        compiler_params=pltpu.CompilerParams(use_tc_tiling_on_sc=False,
                                             needs_layout_passes=False),
        scratch_types=[
            pltpu.VMEM((CN, CH), jnp.int32),
            pltpu.VMEM((CN, CH), jnp.int32),
            pltpu.VMEM((EW,), jnp.int32),
            pltpu.VMEM((2, CH, DH), jnp.float32),
            pltpu.VMEM((N,), jnp.float32),
            pltpu.VMEM_SHARED((N, DH), jnp.float32),
            pltpu.SemaphoreType.DMA,
            pltpu.SemaphoreType.DMA,
        ],
    )
    def sc_kernel(x0_hbm, x1_hbm, src2d, dst2d, dst1d, agg_out, deg_out,
                  src_v, dst_v, dstf_v, rows_v, deg_v,
                  agg_sh, gsem0, gsem1):
        cid = lax.axis_index("c")
        sid = lax.axis_index("s")
        w = cid * NS + sid
        base = sid * CN
        rbase = sid * ROWS_PER_TILE
        zvec = jnp.zeros((L,), jnp.float32)
        ovec = jnp.ones((L,), jnp.float32)

        pltpu.sync_copy(src2d.at[pl.ds(base, CN)], src_v)
        pltpu.sync_copy(dst2d.at[pl.ds(base, CN)], dst_v)
        pltpu.sync_copy(dst1d.at[pl.ds(w * EW, EW)], dstf_v)

        def fill_rows(i, _):
            for k in range(DH // L):
                rows_v[0, i, pl.ds(k * L, L)] = zvec
            return 0
        lax.fori_loop(0, CH, fill_rows, 0)

        def fill_deg(i, _):
            deg_v[pl.ds(i * L, L)] = zvec
            return 0
        lax.fori_loop(0, N // L, fill_deg, 0)

        for t in range(WB):
            pltpu.sync_copy(rows_v.at[0], agg_sh.at[pl.ds(rbase + t * CH, CH)])
        plsc.subcore_barrier()

        def make_loop(x_hbm):
            def pair(p, _):
                j0 = p * 2
                pltpu.async_copy(x_hbm.at[src_v.at[j0 + 1]], rows_v.at[1],
                                 gsem1)
                pltpu.make_async_copy(x_hbm.at[src_v.at[j0]], rows_v.at[0],
                                      gsem0).wait()
                pltpu.sync_copy(rows_v.at[0], agg_sh.at[dst_v.at[j0]],
                                add=True)

                @pl.when(j0 + 2 < CN)
                def _():
                    pltpu.async_copy(x_hbm.at[src_v.at[j0 + 2]],
                                     rows_v.at[0], gsem0)

                pltpu.make_async_copy(x_hbm.at[src_v.at[j0 + 1]],
                                      rows_v.at[1], gsem1).wait()
                pltpu.sync_copy(rows_v.at[1], agg_sh.at[dst_v.at[j0 + 1]],
                                add=True)
                return 0

            pltpu.async_copy(x_hbm.at[src_v.at[0]], rows_v.at[0], gsem0)
            lax.fori_loop(0, CN // 2, pair, 0)

        @pl.when(cid == 0)
        def _():
            make_loop(x0_hbm)

        @pl.when(cid == 1)
        def _():
            make_loop(x1_hbm)

        def count(i, _):
            idx = dstf_v[pl.ds(i * L, L)]
            plsc.addupdate_scatter(deg_v, [idx], ovec)
            return 0
        lax.fori_loop(0, EW // L, count, 0)

        plsc.subcore_barrier()

        for t in range(WB):
            pltpu.sync_copy(agg_sh.at[pl.ds(rbase + t * CH, CH)],
                            agg_out.at[cid, pl.ds(rbase + t * CH, CH)])
        pltpu.sync_copy(deg_v, deg_out.at[w])

    return sc_kernel


_sc_kernel = _make_sc_kernel()


def _tc_body(agg_ref, deg_ref, x_ref, w_ref, b_ref, g_ref, bt_ref, out_ref):
    a = jnp.concatenate([agg_ref[0], agg_ref[1]], axis=1)
    dg = jnp.sum(deg_ref[...], axis=1, keepdims=True)
    dg = jnp.maximum(dg, 1.0)
    an = a / dg
    h = lax.dot_general(an, w_ref[...], (((1,), (0,)), ((), ())),
                        preferred_element_type=jnp.float32,
                        precision=lax.Precision.HIGHEST)
    o = h + b_ref[...] + x_ref[...]
    mu = jnp.mean(o, axis=1, keepdims=True)
    c = o - mu
    var = jnp.mean(c * c, axis=1, keepdims=True)
    y = c * lax.rsqrt(var + 1e-5) * g_ref[...] + bt_ref[...]
    out_ref[...] = 0.5 * y * (1.0 + lax.erf(y * (1.0 / math.sqrt(2.0))))


BN = 1000


def _tc_tail(agg_p, deg_p, x, W, b, g, bt):
    grid = (N // BN,)
    return pl.pallas_call(
        _tc_body,
        grid=grid,
        in_specs=[
            pl.BlockSpec((NC, BN, DH), lambda i: (0, i, 0)),
            pl.BlockSpec((BN, NW), lambda i: (i, 0)),
            pl.BlockSpec((BN, D), lambda i: (i, 0)),
            pl.BlockSpec((D, D), lambda i: (0, 0)),
            pl.BlockSpec((1, D), lambda i: (0, 0)),
            pl.BlockSpec((1, D), lambda i: (0, 0)),
            pl.BlockSpec((1, D), lambda i: (0, 0)),
        ],
        out_specs=pl.BlockSpec((BN, D), lambda i: (i, 0)),
        out_shape=jax.ShapeDtypeStruct((N, D), jnp.float32),
    )(agg_p, deg_p, x, W, b, g, bt)


@jax.jit
def kernel(x, edge_index, W, b, ln_gamma, ln_beta):
    src2d = edge_index[0].reshape(E // CH, CH)
    dst2d = edge_index[1].reshape(E // CH, CH)
    x0 = x[:, :DH]
    x1 = x[:, DH:]
    agg_p, deg_p = _sc_kernel(x0, x1, src2d, dst2d, edge_index[1])
    return _tc_tail(agg_p, deg_p.T, x, W,
                    b.reshape(1, D), ln_gamma.reshape(1, D),
                    ln_beta.reshape(1, D))


# speedup vs baseline: 11.9298x; 1.1564x over previous
import functools
import math

import jax
import jax.numpy as jnp
from jax import lax
from jax.experimental import pallas as pl
from jax.experimental.pallas import tpu as pltpu
from jax.experimental.pallas import tpu_sc as plsc

N = 10000
E = 320000
D = 128
DH = D // 2

NC = 2
NS = 16
NW = NC * NS

CH = 125
CN = (E // CH) // NS
EW = E // NW
ROWS_PER_TILE = N // NS
WB = ROWS_PER_TILE // CH

L = 16


def _make_sc_kernel():
    mesh = plsc.VectorSubcoreMesh(core_axis_name="c", subcore_axis_name="s")

    @functools.partial(
        pl.kernel,
        out_type=[
            jax.ShapeDtypeStruct((NC, N, DH), jnp.float32),
            jax.ShapeDtypeStruct((NW, N), jnp.float32),
        ],
        mesh=mesh,
        compiler_params=pltpu.CompilerParams(use_tc_tiling_on_sc=False,
                                             needs_layout_passes=False),
        scratch_types=[
            pltpu.VMEM((CN, CH), jnp.int32),
            pltpu.VMEM((CN, CH), jnp.int32),
            pltpu.VMEM((EW,), jnp.int32),
            pltpu.VMEM((2, CH, DH), jnp.float32),
            pltpu.VMEM((N,), jnp.float32),
            pltpu.VMEM_SHARED((N, DH), jnp.float32),
            pltpu.SemaphoreType.DMA,
            pltpu.SemaphoreType.DMA,
        ],
    )
    def sc_kernel(x2_hbm, srce2d, srco2d, dst2d, dst1d, agg_out, deg_out,
                  src_v, dst_v, dstf_v, rows_v, deg_v,
                  agg_sh, gsem0, gsem1):
        cid = lax.axis_index("c")
        sid = lax.axis_index("s")
        w = cid * NS + sid
        base = sid * CN
        rbase = sid * ROWS_PER_TILE
        zvec = jnp.zeros((L,), jnp.float32)
        ovec = jnp.ones((L,), jnp.float32)

        @pl.when(cid == 0)
        def _():
            pltpu.sync_copy(srce2d.at[pl.ds(base, CN)], src_v)

        @pl.when(cid == 1)
        def _():
            pltpu.sync_copy(srco2d.at[pl.ds(base, CN)], src_v)

        pltpu.sync_copy(dst2d.at[pl.ds(base, CN)], dst_v)
        pltpu.sync_copy(dst1d.at[pl.ds(w * EW, EW)], dstf_v)

        def fill_rows(i, _):
            for k in range(DH // L):
                rows_v[0, i, pl.ds(k * L, L)] = zvec
            return 0
        lax.fori_loop(0, CH, fill_rows, 0)

        def fill_deg(i, _):
            deg_v[pl.ds(i * L, L)] = zvec
            return 0
        lax.fori_loop(0, N // L, fill_deg, 0)

        for t in range(WB):
            pltpu.sync_copy(rows_v.at[0], agg_sh.at[pl.ds(rbase + t * CH, CH)])
        plsc.subcore_barrier()

        def pair(p, _):
            j0 = p * 2
            pltpu.async_copy(x2_hbm.at[src_v.at[j0 + 1]], rows_v.at[1],
                             gsem1)
            pltpu.make_async_copy(x2_hbm.at[src_v.at[j0]], rows_v.at[0],
                                  gsem0).wait()
            pltpu.sync_copy(rows_v.at[0], agg_sh.at[dst_v.at[j0]],
                            add=True)

            @pl.when(j0 + 2 < CN)
            def _():
                pltpu.async_copy(x2_hbm.at[src_v.at[j0 + 2]],
                                 rows_v.at[0], gsem0)

            pltpu.make_async_copy(x2_hbm.at[src_v.at[j0 + 1]],
                                  rows_v.at[1], gsem1).wait()
            pltpu.sync_copy(rows_v.at[1], agg_sh.at[dst_v.at[j0 + 1]],
                            add=True)
            return 0

        pltpu.async_copy(x2_hbm.at[src_v.at[0]], rows_v.at[0], gsem0)
        lax.fori_loop(0, CN // 2, pair, 0)

        def count(i, _):
            idx = dstf_v[pl.ds(i * L, L)]
            plsc.addupdate_scatter(deg_v, [idx], ovec)
            return 0
        lax.fori_loop(0, EW // L, count, 0)

        plsc.subcore_barrier()

        for t in range(WB):
            pltpu.sync_copy(agg_sh.at[pl.ds(rbase + t * CH, CH)],
                            agg_out.at[cid, pl.ds(rbase + t * CH, CH)])
        pltpu.sync_copy(deg_v, deg_out.at[w])

    return sc_kernel


_sc_kernel = _make_sc_kernel()


def _tc_body(agg_ref, deg_ref, x_ref, w_ref, b_ref, g_ref, bt_ref, out_ref):
    a = jnp.concatenate([agg_ref[0], agg_ref[1]], axis=1)
    dg = jnp.sum(deg_ref[...], axis=0)[:, None]
    dg = jnp.maximum(dg, 1.0)
    an = a / dg
    h = lax.dot_general(an, w_ref[...], (((1,), (0,)), ((), ())),
                        preferred_element_type=jnp.float32,
                        precision=lax.Precision.HIGHEST)
    o = h + b_ref[...] + x_ref[...]
    mu = jnp.mean(o, axis=1, keepdims=True)
    c = o - mu
    var = jnp.mean(c * c, axis=1, keepdims=True)
    y = c * lax.rsqrt(var + 1e-5) * g_ref[...] + bt_ref[...]
    out_ref[...] = 0.5 * y * (1.0 + lax.erf(y * (1.0 / math.sqrt(2.0))))


def _tc_tail(agg_p, deg_p, x, W, b, g, bt):
    return pl.pallas_call(
        _tc_body,
        out_shape=jax.ShapeDtypeStruct((N, D), jnp.float32),
    )(agg_p, deg_p, x, W, b, g, bt)


@jax.jit
def kernel(x, edge_index, W, b, ln_gamma, ln_beta):
    src2 = edge_index[0] * 2
    src2d_even = src2.reshape(E // CH, CH)
    src2d_odd = (src2 + 1).reshape(E // CH, CH)
    dst2d = edge_index[1].reshape(E // CH, CH)
    x2 = x.reshape(2 * N, DH)
    agg_p, deg_p = _sc_kernel(x2, src2d_even, src2d_odd, dst2d, edge_index[1])
    return _tc_tail(agg_p, deg_p, x, W,
                    b.reshape(1, D), ln_gamma.reshape(1, D),
                    ln_beta.reshape(1, D))


# speedup vs baseline: 12.8409x; 1.0764x over previous
import functools
import math

import jax
import jax.numpy as jnp
from jax import lax
from jax.experimental import pallas as pl
from jax.experimental.pallas import tpu as pltpu
from jax.experimental.pallas import tpu_sc as plsc

N = 10000
E = 320000
D = 128
DH = D // 2

NC = 2
NS = 16
NW = NC * NS

CH = 125
CN = (E // CH) // NS
EW = E // NW
ROWS_PER_TILE = N // NS
WB = ROWS_PER_TILE // CH

L = 16


def _make_sc_kernel():
    mesh = plsc.VectorSubcoreMesh(core_axis_name="c", subcore_axis_name="s")

    @functools.partial(
        pl.kernel,
        out_type=[
            jax.ShapeDtypeStruct((NC, N, DH), jnp.float32),
            jax.ShapeDtypeStruct((NW, N), jnp.float32),
        ],
        mesh=mesh,
        compiler_params=pltpu.CompilerParams(use_tc_tiling_on_sc=False,
                                             needs_layout_passes=False),
        scratch_types=[
            pltpu.VMEM((CN, CH), jnp.int32),
            pltpu.VMEM((CN, CH), jnp.int32),
            pltpu.VMEM((4, CH, DH), jnp.float32),
            pltpu.VMEM((N,), jnp.float32),
            pltpu.VMEM_SHARED((N, DH), jnp.float32),
            pltpu.SemaphoreType.DMA,
            pltpu.SemaphoreType.DMA,
            pltpu.SemaphoreType.DMA,
            pltpu.SemaphoreType.DMA,
            pltpu.SemaphoreType.DMA,
            pltpu.SemaphoreType.DMA,
            pltpu.SemaphoreType.DMA,
            pltpu.SemaphoreType.DMA,
            pltpu.SemaphoreType.DMA,
        ],
    )
    def sc_kernel(x2_hbm, srce2d, srco2d, dst2d, agg_out, deg_out,
                  src_v, dst_v, rows_v, deg_v, agg_sh,
                  g0, g1, g2, g3, s0, s1, s2, s3, stsem):
        cid = lax.axis_index("c")
        sid = lax.axis_index("s")
        w = cid * NS + sid
        base = sid * CN
        rbase = sid * ROWS_PER_TILE
        zvec = jnp.zeros((L,), jnp.float32)
        ovec = jnp.ones((L,), jnp.float32)

        gsems = [g0, g1, g2, g3]
        ssems = [s0, s1, s2, s3]

        @pl.when(cid == 0)
        def _():
            pltpu.async_copy(srce2d.at[pl.ds(base, CN)], src_v, stsem)

        @pl.when(cid == 1)
        def _():
            pltpu.async_copy(srco2d.at[pl.ds(base, CN)], src_v, stsem)

        pltpu.async_copy(dst2d.at[pl.ds(base, CN)], dst_v, g0)

        def fill_rows(i, _):
            for k in range(DH // L):
                rows_v[0, i, pl.ds(k * L, L)] = zvec
            return 0
        lax.fori_loop(0, CH, fill_rows, 0)

        def fill_deg(i, _):
            deg_v[pl.ds(i * L, L)] = zvec
            return 0
        lax.fori_loop(0, N // L, fill_deg, 0)

        for t in range(WB):
            pltpu.sync_copy(rows_v.at[0], agg_sh.at[pl.ds(rbase + t * CH, CH)])
        pltpu.make_async_copy(srce2d.at[pl.ds(base, CN)], src_v, stsem).wait()
        pltpu.make_async_copy(dst2d.at[pl.ds(base, CN)], dst_v, g0).wait()
        plsc.subcore_barrier()

        def gather(j, k):
            pltpu.async_copy(x2_hbm.at[src_v.at[j]], rows_v.at[k], gsems[k])

        def wait_gather(j, k):
            pltpu.make_async_copy(x2_hbm.at[src_v.at[j]], rows_v.at[k],
                                  gsems[k]).wait()

        def scatter(j, k):
            pltpu.async_copy(rows_v.at[k], agg_sh.at[dst_v.at[j]], ssems[k],
                             add=True)

        def wait_scatter(j, k):
            pltpu.make_async_copy(rows_v.at[k], agg_sh.at[dst_v.at[j]],
                                  ssems[k]).wait()

        tail_mask = lax.iota(jnp.int32, L) >= (L - (CH - (CH // L) * L))

        def count_chunk(j):
            for q in range(CH // L):
                idx = dst_v[j, pl.ds(q * L, L)]
                plsc.addupdate_scatter(deg_v, [idx], ovec)
            idx = dst_v[j, pl.ds(CH - L, L)]
            plsc.addupdate_scatter(deg_v, [idx], ovec, mask=tail_mask)

        def quad(t, _):
            for q in range(4):
                j = t * 4 + q
                wait_gather(j, q)
                scatter(j, q)
                count_chunk(j)

                @pl.when(j >= 2)
                def _():
                    wait_scatter(j - 2, (q + 2) % 4)

                @pl.when(j + 2 < CN)
                def _():
                    gather(j + 2, (q + 2) % 4)
            return 0

        gather(0, 0)
        gather(1, 1)
        lax.fori_loop(0, CN // 4, quad, 0)
        wait_scatter(CN - 2, 2)
        wait_scatter(CN - 1, 3)

        plsc.subcore_barrier()

        for t in range(WB):
            pltpu.sync_copy(agg_sh.at[pl.ds(rbase + t * CH, CH)],
                            agg_out.at[cid, pl.ds(rbase + t * CH, CH)])
        pltpu.sync_copy(deg_v, deg_out.at[w])

    return sc_kernel


_sc_kernel = _make_sc_kernel()


def _tc_body(agg_ref, deg_ref, x_ref, w_ref, b_ref, g_ref, bt_ref, out_ref):
    a = jnp.concatenate([agg_ref[0], agg_ref[1]], axis=1)
    dg = jnp.sum(deg_ref[...], axis=0)[:, None]
    dg = jnp.maximum(dg, 1.0)
    an = a / dg
    h = lax.dot_general(an, w_ref[...], (((1,), (0,)), ((), ())),
                        preferred_element_type=jnp.float32,
                        precision=lax.Precision.HIGHEST)
    o = h + b_ref[...] + x_ref[...]
    mu = jnp.mean(o, axis=1, keepdims=True)
    c = o - mu
    var = jnp.mean(c * c, axis=1, keepdims=True)
    y = c * lax.rsqrt(var + 1e-5) * g_ref[...] + bt_ref[...]
    out_ref[...] = 0.5 * y * (1.0 + lax.erf(y * (1.0 / math.sqrt(2.0))))


def _tc_tail(agg_p, deg_p, x, W, b, g, bt):
    return pl.pallas_call(
        _tc_body,
        out_shape=jax.ShapeDtypeStruct((N, D), jnp.float32),
    )(agg_p, deg_p, x, W, b, g, bt)


@jax.jit
def kernel(x, edge_index, W, b, ln_gamma, ln_beta):
    src2 = edge_index[0] * 2
    src2d_even = src2.reshape(E // CH, CH)
    src2d_odd = (src2 + 1).reshape(E // CH, CH)
    dst2d = edge_index[1].reshape(E // CH, CH)
    x2 = x.reshape(2 * N, DH)
    agg_p, deg_p = _sc_kernel(x2, src2d_even, src2d_odd, dst2d)
    return _tc_tail(agg_p, deg_p, x, W,
                    b.reshape(1, D), ln_gamma.reshape(1, D),
                    ln_beta.reshape(1, D))
